# baseline (speedup 1.0000x reference)
import functools

import jax
import jax.numpy as jnp
from jax import lax
from jax.experimental import pallas as pl
from jax.experimental.pallas import tpu as pltpu
from jax.experimental.pallas import tpu_sc as plsc

N_NODES = 10000
N_EDGES = 320000
D_NODE = 128
D_EDGE = 16

FOLD = 8
EROWS = N_EDGES // FOLD
PROJ_BLOCK = 800
MLP_BLOCK = 1000

NC = 2
NS = 16
CH = 80
EPC = N_EDGES // NC
EPT = EPC // NS
NCH = EPT // CH
N_PAD = 10240
ROWS_PT = N_PAD // NS


def _proj_body(eat_ref, w_ref, b_ref, out_ref):
    out_ref[...] = (
        lax.dot_general(eat_ref[...], w_ref[...],
                        dimension_numbers=(((0,), (0,)), ((), ())),
                        preferred_element_type=jnp.float32)
        + b_ref[...]
    )


def _mlp_body(x_ref, p_ref, w1_ref, b1_ref, w2_ref, b2_ref, out_ref):
    h = x_ref[...] + p_ref[0] + p_ref[1]
    h = jnp.maximum(
        jnp.dot(h, w1_ref[...], preferred_element_type=jnp.float32) + b1_ref[...],
        0.0,
    )
    out_ref[...] = (
        jnp.dot(h, w2_ref[...], preferred_element_type=jnp.float32) + b2_ref[...]
    )


def _sc_aggregate(x_hbm, src_hbm, dst_hbm, e_hbm, zero_hbm, out_hbm,
                  src_v0, src_v1, dst_v0, dst_v1, e_v, xr_v, aggr_sh,
                  e_sem0, e_sem1, g_sem0, g_sem1, i_sem0, i_sem1):
    core = lax.axis_index("c")
    sub = lax.axis_index("s")
    src_vs = (src_v0, src_v1)
    dst_vs = (dst_v0, dst_v1)
    e_sems = (e_sem0, e_sem1)
    g_sems = (g_sem0, g_sem1)
    i_sems = (i_sem0, i_sem1)

    row0 = sub * ROWS_PT
    pltpu.sync_copy(zero_hbm.at[pl.ds(row0, ROWS_PT)],
                    aggr_sh.at[pl.ds(row0, ROWS_PT)])
    plsc.subcore_barrier()

    base0 = core * EPC + sub * EPT

    def idx_load(ci, s):
        base = base0 + ci * CH
        pltpu.async_copy(src_hbm.at[pl.ds(base, CH)], src_vs[s], i_sems[s])
        pltpu.async_copy(dst_hbm.at[pl.ds(base, CH)], dst_vs[s], i_sems[s])

    def streams(ci, s):
        base = base0 + ci * CH
        pltpu.make_async_copy(src_hbm.at[pl.ds(base, CH)], src_vs[s],
                              i_sems[s]).wait()
        pltpu.make_async_copy(dst_hbm.at[pl.ds(base, CH)], dst_vs[s],
                              i_sems[s]).wait()
        pltpu.async_copy(e_hbm.at[pl.ds(base, CH)], e_v.at[s], e_sems[s])
        pltpu.async_copy(x_hbm.at[src_vs[s]], xr_v.at[s], g_sems[s])

    def process(ci, s):
        base = base0 + ci * CH
        pltpu.make_async_copy(e_hbm.at[pl.ds(base, CH)], e_v.at[s],
                              e_sems[s]).wait()
        pltpu.make_async_copy(x_hbm.at[src_vs[s]], xr_v.at[s],
                              g_sems[s]).wait()

        @plsc.parallel_loop(0, CH, 1, unroll=4)
        def row_body(r):
            for j in range(D_NODE // 16):
                sl = pl.ds(j * 16, 16)
                e_v[s, r, sl] = jnp.maximum(e_v[s, r, sl] + xr_v[s, r, sl],
                                            0.0)
        pltpu.sync_copy(e_v.at[s], aggr_sh.at[dst_vs[s]], add=True)

    idx_load(0, 0)
    streams(0, 0)
    idx_load(1, 1)

    def pair_body(k2, c):
        i = 2 * k2
        streams(i + 1, 1)
        process(i, 0)
        idx_load(i + 2, 0)
        streams(i + 2, 0)
        process(i + 1, 1)
        idx_load(jnp.minimum(i + 3, NCH - 1), 1)
        return c

    lax.fori_loop(0, NCH // 2, pair_body, 0)
    process(NCH - 1, 0)
    pltpu.make_async_copy(src_hbm.at[pl.ds(base0, CH)], src_vs[1],
                          i_sems[1]).wait()
    pltpu.make_async_copy(dst_hbm.at[pl.ds(base0, CH)], dst_vs[1],
                          i_sems[1]).wait()
    plsc.subcore_barrier()

    pltpu.sync_copy(aggr_sh.at[pl.ds(row0, ROWS_PT)],
                    out_hbm.at[core, pl.ds(row0, ROWS_PT)])


def kernel(x, edge_index, edge_attr, We, be, W1, b1, W2, b2):
    ei = edge_index.astype(jnp.int32)

    eat = edge_attr.T
    e = pl.pallas_call(
        _proj_body,
        grid=(N_EDGES // 6400,),
        in_specs=[
            pl.BlockSpec((D_EDGE, 6400), lambda i: (0, i)),
            pl.BlockSpec((D_EDGE, D_NODE), lambda i: (0, 0)),
            pl.BlockSpec((1, D_NODE), lambda i: (0, 0)),
        ],
        out_specs=pl.BlockSpec((6400, D_NODE), lambda i: (i, 0)),
        out_shape=jax.ShapeDtypeStruct((N_EDGES, D_NODE), jnp.float32),
    )(eat, We, be.reshape(1, D_NODE))

    zero = jnp.zeros((N_PAD, D_NODE), jnp.float32)
    partials = pl.kernel(
        _sc_aggregate,
        mesh=plsc.VectorSubcoreMesh(core_axis_name="c", subcore_axis_name="s"),
        out_type=jax.ShapeDtypeStruct((NC, N_PAD, D_NODE), jnp.float32),
        scratch_types=[
            pltpu.VMEM((CH,), jnp.int32),
            pltpu.VMEM((CH,), jnp.int32),
            pltpu.VMEM((CH,), jnp.int32),
            pltpu.VMEM((CH,), jnp.int32),
            pltpu.VMEM((2, CH, D_NODE), jnp.float32),
            pltpu.VMEM((2, CH, D_NODE), jnp.float32),
            pltpu.VMEM_SHARED((N_PAD, D_NODE), jnp.float32),
            pltpu.SemaphoreType.DMA,
            pltpu.SemaphoreType.DMA,
            pltpu.SemaphoreType.DMA,
            pltpu.SemaphoreType.DMA,
            pltpu.SemaphoreType.DMA,
            pltpu.SemaphoreType.DMA,
        ],
    )(x, ei[0], ei[1], e, zero)
    partials = partials[:, :N_NODES, :]

    out = pl.pallas_call(
        _mlp_body,
        grid=(N_NODES // MLP_BLOCK,),
        in_specs=[
            pl.BlockSpec((MLP_BLOCK, D_NODE), lambda i: (i, 0)),
            pl.BlockSpec((NC, MLP_BLOCK, D_NODE), lambda i: (0, i, 0)),
            pl.BlockSpec((D_NODE, D_NODE), lambda i: (0, 0)),
            pl.BlockSpec((1, D_NODE), lambda i: (0, 0)),
            pl.BlockSpec((D_NODE, D_NODE), lambda i: (0, 0)),
            pl.BlockSpec((1, D_NODE), lambda i: (0, 0)),
        ],
        out_specs=pl.BlockSpec((MLP_BLOCK, D_NODE), lambda i: (i, 0)),
        out_shape=jax.ShapeDtypeStruct((N_NODES, D_NODE), jnp.float32),
    )(x, partials, W1, b1.reshape(1, D_NODE), W2, b2.reshape(1, D_NODE))
    return out


# speedup vs baseline: 1.0036x; 1.0036x over previous
import functools

import jax
import jax.numpy as jnp
from jax import lax
from jax.experimental import pallas as pl
from jax.experimental.pallas import tpu as pltpu
from jax.experimental.pallas import tpu_sc as plsc

N_NODES = 10000
N_EDGES = 320000
D_NODE = 128
D_EDGE = 16

FOLD = 8
EROWS = N_EDGES // FOLD
PROJ_BLOCK = 800
MLP_BLOCK = 1000

NC = 2
NS = 16
CH = 80
EPC = N_EDGES // NC
EPT = EPC // NS
NCH = EPT // CH
N_PAD = 10240
ROWS_PT = N_PAD // NS


def _proj_body(eat_ref, w_ref, b_ref, out_ref):
    out_ref[...] = (
        lax.dot_general(eat_ref[...], w_ref[...],
                        dimension_numbers=(((0,), (0,)), ((), ())),
                        preferred_element_type=jnp.float32)
        + b_ref[...]
    )


def _mlp_body(x_ref, p_ref, w1_ref, b1_ref, w2_ref, b2_ref, out_ref):
    h = x_ref[...] + p_ref[0] + p_ref[1]
    h = jnp.maximum(
        jnp.dot(h, w1_ref[...], preferred_element_type=jnp.float32) + b1_ref[...],
        0.0,
    )
    out_ref[...] = (
        jnp.dot(h, w2_ref[...], preferred_element_type=jnp.float32) + b2_ref[...]
    )


def _sc_aggregate(x_hbm, src_hbm, dst_hbm, e_hbm, zero_hbm, out_hbm,
                  src_v0, src_v1, dst_v0, dst_v1, e_v, xr_v, aggr_sh,
                  e_sem0, e_sem1, g_sem0, g_sem1, i_sem0, i_sem1):
    core = lax.axis_index("c")
    sub = lax.axis_index("s")
    src_vs = (src_v0, src_v1)
    dst_vs = (dst_v0, dst_v1)
    e_sems = (e_sem0, e_sem1)
    g_sems = (g_sem0, g_sem1)
    i_sems = (i_sem0, i_sem1)

    row0 = sub * ROWS_PT
    pltpu.sync_copy(zero_hbm.at[pl.ds(row0, ROWS_PT)],
                    aggr_sh.at[pl.ds(row0, ROWS_PT)])
    plsc.subcore_barrier()

    base0 = core * EPC + sub * EPT

    def idx_load(ci, s):
        base = base0 + ci * CH
        pltpu.async_copy(src_hbm.at[pl.ds(base, CH)], src_vs[s], i_sems[s])
        pltpu.async_copy(dst_hbm.at[pl.ds(base, CH)], dst_vs[s], i_sems[s])

    def streams(ci, s):
        base = base0 + ci * CH
        pltpu.make_async_copy(src_hbm.at[pl.ds(base, CH)], src_vs[s],
                              i_sems[s]).wait()
        pltpu.make_async_copy(dst_hbm.at[pl.ds(base, CH)], dst_vs[s],
                              i_sems[s]).wait()
        pltpu.async_copy(e_hbm.at[pl.ds(base, CH)], e_v.at[s], e_sems[s])
        pltpu.async_copy(x_hbm.at[src_vs[s]], xr_v.at[s], g_sems[s])

    def process(ci, s):
        base = base0 + ci * CH
        pltpu.make_async_copy(e_hbm.at[pl.ds(base, CH)], e_v.at[s],
                              e_sems[s]).wait()
        pltpu.make_async_copy(x_hbm.at[src_vs[s]], xr_v.at[s],
                              g_sems[s]).wait()

        def row_body(r, c2):
            for j in range(D_NODE // 16):
                sl = pl.ds(j * 16, 16)
                e_v[s, r, sl] = jnp.maximum(e_v[s, r, sl] + xr_v[s, r, sl],
                                            0.0)
            return c2

        lax.fori_loop(0, CH, row_body, 0)
        pltpu.sync_copy(e_v.at[s], aggr_sh.at[dst_vs[s]], add=True)

    idx_load(0, 0)
    streams(0, 0)
    idx_load(1, 1)

    def pair_body(k2, c):
        i = 2 * k2
        streams(i + 1, 1)
        process(i, 0)
        idx_load(i + 2, 0)
        streams(i + 2, 0)
        process(i + 1, 1)
        idx_load(jnp.minimum(i + 3, NCH - 1), 1)
        return c

    lax.fori_loop(0, NCH // 2, pair_body, 0)
    process(NCH - 1, 0)
    pltpu.make_async_copy(src_hbm.at[pl.ds(base0, CH)], src_vs[1],
                          i_sems[1]).wait()
    pltpu.make_async_copy(dst_hbm.at[pl.ds(base0, CH)], dst_vs[1],
                          i_sems[1]).wait()
    plsc.subcore_barrier()

    pltpu.sync_copy(aggr_sh.at[pl.ds(row0, ROWS_PT)],
                    out_hbm.at[core, pl.ds(row0, ROWS_PT)])


def kernel(x, edge_index, edge_attr, We, be, W1, b1, W2, b2):
    ei = edge_index.astype(jnp.int32)

    eat = edge_attr.T
    e = pl.pallas_call(
        _proj_body,
        grid=(N_EDGES // 6400,),
        in_specs=[
            pl.BlockSpec((D_EDGE, 6400), lambda i: (0, i)),
            pl.BlockSpec((D_EDGE, D_NODE), lambda i: (0, 0)),
            pl.BlockSpec((1, D_NODE), lambda i: (0, 0)),
        ],
        out_specs=pl.BlockSpec((6400, D_NODE), lambda i: (i, 0)),
        out_shape=jax.ShapeDtypeStruct((N_EDGES, D_NODE), jnp.float32),
    )(eat, We, be.reshape(1, D_NODE))

    zero = jnp.zeros((N_PAD, D_NODE), jnp.float32)
    partials = pl.kernel(
        _sc_aggregate,
        mesh=plsc.VectorSubcoreMesh(core_axis_name="c", subcore_axis_name="s"),
        out_type=jax.ShapeDtypeStruct((NC, N_PAD, D_NODE), jnp.float32),
        scratch_types=[
            pltpu.VMEM((CH,), jnp.int32),
            pltpu.VMEM((CH,), jnp.int32),
            pltpu.VMEM((CH,), jnp.int32),
            pltpu.VMEM((CH,), jnp.int32),
            pltpu.VMEM((2, CH, D_NODE), jnp.float32),
            pltpu.VMEM((2, CH, D_NODE), jnp.float32),
            pltpu.VMEM_SHARED((N_PAD, D_NODE), jnp.float32),
            pltpu.SemaphoreType.DMA,
            pltpu.SemaphoreType.DMA,
            pltpu.SemaphoreType.DMA,
            pltpu.SemaphoreType.DMA,
            pltpu.SemaphoreType.DMA,
            pltpu.SemaphoreType.DMA,
        ],
    )(x, ei[0], ei[1], e, zero)
    partials = partials[:, :N_NODES, :]

    out = pl.pallas_call(
        _mlp_body,
        grid=(N_NODES // MLP_BLOCK,),
        in_specs=[
            pl.BlockSpec((MLP_BLOCK, D_NODE), lambda i: (i, 0)),
            pl.BlockSpec((NC, MLP_BLOCK, D_NODE), lambda i: (0, i, 0)),
            pl.BlockSpec((D_NODE, D_NODE), lambda i: (0, 0)),
            pl.BlockSpec((1, D_NODE), lambda i: (0, 0)),
            pl.BlockSpec((D_NODE, D_NODE), lambda i: (0, 0)),
            pl.BlockSpec((1, D_NODE), lambda i: (0, 0)),
        ],
        out_specs=pl.BlockSpec((MLP_BLOCK, D_NODE), lambda i: (i, 0)),
        out_shape=jax.ShapeDtypeStruct((N_NODES, D_NODE), jnp.float32),
    )(x, partials, W1, b1.reshape(1, D_NODE), W2, b2.reshape(1, D_NODE))
    return out


# speedup vs baseline: 1.0394x; 1.0357x over previous
import functools

import jax
import jax.numpy as jnp
from jax import lax
from jax.experimental import pallas as pl
from jax.experimental.pallas import tpu as pltpu
from jax.experimental.pallas import tpu_sc as plsc

N_NODES = 10000
N_EDGES = 320000
D_NODE = 128
D_EDGE = 16

FOLD = 8
EROWS = N_EDGES // FOLD
PROJ_BLOCK = 800
MLP_BLOCK = 1000

NC = 2
NS = 16
CH = 80
EPC = N_EDGES // NC
EPT = EPC // NS
NCH = EPT // CH
N_PAD = 10240
ROWS_PT = N_PAD // NS


def _proj_body(eat_ref, w_ref, b_ref, out_ref):
    out_ref[...] = (
        lax.dot_general(eat_ref[...], w_ref[...],
                        dimension_numbers=(((0,), (0,)), ((), ())),
                        preferred_element_type=jnp.float32)
        + b_ref[...]
    )


def _mlp_body(x_ref, p_ref, w1_ref, b1_ref, w2_ref, b2_ref, out_ref):
    h = x_ref[...] + p_ref[0] + p_ref[1]
    h = jnp.maximum(
        jnp.dot(h, w1_ref[...], preferred_element_type=jnp.float32) + b1_ref[...],
        0.0,
    )
    out_ref[...] = (
        jnp.dot(h, w2_ref[...], preferred_element_type=jnp.float32) + b2_ref[...]
    )


def _sc_aggregate(x_hbm, src_hbm, dst_hbm, e_hbm, out_hbm,
                  src_v0, src_v1, dst_v0, dst_v1, e_v, xr_v, aggr_sh,
                  e_sem0, e_sem1, g_sem0, g_sem1, i_sem0, i_sem1):
    core = lax.axis_index("c")
    sub = lax.axis_index("s")
    src_vs = (src_v0, src_v1)
    dst_vs = (dst_v0, dst_v1)
    e_sems = (e_sem0, e_sem1)
    g_sems = (g_sem0, g_sem1)
    i_sems = (i_sem0, i_sem1)

    row0 = sub * ROWS_PT

    def zrow(r, c2):
        for j in range(D_NODE // 16):
            e_v[0, r, pl.ds(j * 16, 16)] = jnp.zeros((16,), jnp.float32)
        return c2

    lax.fori_loop(0, CH, zrow, 0)
    for b in range(ROWS_PT // CH):
        pltpu.sync_copy(e_v.at[0], aggr_sh.at[pl.ds(row0 + b * CH, CH)])
    plsc.subcore_barrier()

    base0 = core * EPC + sub * EPT

    def idx_load(ci, s):
        base = base0 + ci * CH
        pltpu.async_copy(src_hbm.at[pl.ds(base, CH)], src_vs[s], i_sems[s])
        pltpu.async_copy(dst_hbm.at[pl.ds(base, CH)], dst_vs[s], i_sems[s])

    def streams(ci, s):
        base = base0 + ci * CH
        pltpu.make_async_copy(src_hbm.at[pl.ds(base, CH)], src_vs[s],
                              i_sems[s]).wait()
        pltpu.make_async_copy(dst_hbm.at[pl.ds(base, CH)], dst_vs[s],
                              i_sems[s]).wait()
        pltpu.async_copy(e_hbm.at[pl.ds(base, CH)], e_v.at[s], e_sems[s])
        pltpu.async_copy(x_hbm.at[src_vs[s]], xr_v.at[s], g_sems[s])

    def process(ci, s):
        base = base0 + ci * CH
        pltpu.make_async_copy(e_hbm.at[pl.ds(base, CH)], e_v.at[s],
                              e_sems[s]).wait()
        pltpu.make_async_copy(x_hbm.at[src_vs[s]], xr_v.at[s],
                              g_sems[s]).wait()

        def row_body(r, c2):
            for j in range(D_NODE // 16):
                sl = pl.ds(j * 16, 16)
                e_v[s, r, sl] = jnp.maximum(e_v[s, r, sl] + xr_v[s, r, sl],
                                            0.0)
            return c2

        lax.fori_loop(0, CH, row_body, 0)
        pltpu.sync_copy(e_v.at[s], aggr_sh.at[dst_vs[s]], add=True)

    idx_load(0, 0)
    streams(0, 0)
    idx_load(1, 1)

    def pair_body(k2, c):
        i = 2 * k2
        streams(i + 1, 1)
        process(i, 0)
        idx_load(i + 2, 0)
        streams(i + 2, 0)
        process(i + 1, 1)
        idx_load(jnp.minimum(i + 3, NCH - 1), 1)
        return c

    lax.fori_loop(0, NCH // 2, pair_body, 0)
    process(NCH - 1, 0)
    pltpu.make_async_copy(src_hbm.at[pl.ds(base0, CH)], src_vs[1],
                          i_sems[1]).wait()
    pltpu.make_async_copy(dst_hbm.at[pl.ds(base0, CH)], dst_vs[1],
                          i_sems[1]).wait()
    plsc.subcore_barrier()

    pltpu.sync_copy(aggr_sh.at[pl.ds(row0, ROWS_PT)],
                    out_hbm.at[core, pl.ds(row0, ROWS_PT)])


def kernel(x, edge_index, edge_attr, We, be, W1, b1, W2, b2):
    ei = edge_index.astype(jnp.int32)

    eat = edge_attr.T
    e = pl.pallas_call(
        _proj_body,
        grid=(N_EDGES // 6400,),
        in_specs=[
            pl.BlockSpec((D_EDGE, 6400), lambda i: (0, i)),
            pl.BlockSpec((D_EDGE, D_NODE), lambda i: (0, 0)),
            pl.BlockSpec((1, D_NODE), lambda i: (0, 0)),
        ],
        out_specs=pl.BlockSpec((6400, D_NODE), lambda i: (i, 0)),
        out_shape=jax.ShapeDtypeStruct((N_EDGES, D_NODE), jnp.float32),
    )(eat, We, be.reshape(1, D_NODE))

    partials = pl.kernel(
        _sc_aggregate,
        mesh=plsc.VectorSubcoreMesh(core_axis_name="c", subcore_axis_name="s"),
        out_type=jax.ShapeDtypeStruct((NC, N_PAD, D_NODE), jnp.float32),
        scratch_types=[
            pltpu.VMEM((CH,), jnp.int32),
            pltpu.VMEM((CH,), jnp.int32),
            pltpu.VMEM((CH,), jnp.int32),
            pltpu.VMEM((CH,), jnp.int32),
            pltpu.VMEM((2, CH, D_NODE), jnp.float32),
            pltpu.VMEM((2, CH, D_NODE), jnp.float32),
            pltpu.VMEM_SHARED((N_PAD, D_NODE), jnp.float32),
            pltpu.SemaphoreType.DMA,
            pltpu.SemaphoreType.DMA,
            pltpu.SemaphoreType.DMA,
            pltpu.SemaphoreType.DMA,
            pltpu.SemaphoreType.DMA,
            pltpu.SemaphoreType.DMA,
        ],
    )(x, ei[0], ei[1], e)

    out = pl.pallas_call(
        _mlp_body,
        grid=(N_NODES // MLP_BLOCK,),
        in_specs=[
            pl.BlockSpec((MLP_BLOCK, D_NODE), lambda i: (i, 0)),
            pl.BlockSpec((NC, MLP_BLOCK, D_NODE), lambda i: (0, i, 0)),
            pl.BlockSpec((D_NODE, D_NODE), lambda i: (0, 0)),
            pl.BlockSpec((1, D_NODE), lambda i: (0, 0)),
            pl.BlockSpec((D_NODE, D_NODE), lambda i: (0, 0)),
            pl.BlockSpec((1, D_NODE), lambda i: (0, 0)),
        ],
        out_specs=pl.BlockSpec((MLP_BLOCK, D_NODE), lambda i: (i, 0)),
        out_shape=jax.ShapeDtypeStruct((N_NODES, D_NODE), jnp.float32),
    )(x, partials, W1, b1.reshape(1, D_NODE), W2, b2.reshape(1, D_NODE))
    return out


# speedup vs baseline: 1.0949x; 1.0535x over previous
import functools

import jax
import jax.numpy as jnp
from jax import lax
from jax.experimental import pallas as pl
from jax.experimental.pallas import tpu as pltpu
from jax.experimental.pallas import tpu_sc as plsc

N_NODES = 10000
N_EDGES = 320000
D_NODE = 128
D_EDGE = 16

FOLD = 8
EROWS = N_EDGES // FOLD
PROJ_BLOCK = 800
MLP_BLOCK = 2000

NC = 2
NS = 16
CH = 80
EPC = N_EDGES // NC
EPT = EPC // NS
NCH = EPT // CH
N_PAD = 10240
ROWS_PT = N_PAD // NS


def _proj_body(eat_ref, w_ref, b_ref, out_ref):
    out_ref[...] = (
        lax.dot_general(eat_ref[...], w_ref[...],
                        dimension_numbers=(((0,), (0,)), ((), ())),
                        preferred_element_type=jnp.float32)
        + b_ref[...]
    )


def _mlp_body(x_ref, p_ref, w1_ref, b1_ref, w2_ref, b2_ref, out_ref):
    h = x_ref[...] + p_ref[0] + p_ref[1]
    h = jnp.maximum(
        jnp.dot(h, w1_ref[...], preferred_element_type=jnp.float32) + b1_ref[...],
        0.0,
    )
    out_ref[...] = (
        jnp.dot(h, w2_ref[...], preferred_element_type=jnp.float32) + b2_ref[...]
    )


def _sc_aggregate(x_hbm, src_hbm, dst_hbm, e_hbm, out_hbm,
                  src_v0, src_v1, dst_v0, dst_v1, e_v, xr_v, aggr_sh,
                  e_sem0, e_sem1, g_sem0, g_sem1, i_sem0, i_sem1):
    core = lax.axis_index("c")
    sub = lax.axis_index("s")
    src_vs = (src_v0, src_v1)
    dst_vs = (dst_v0, dst_v1)
    e_sems = (e_sem0, e_sem1)
    g_sems = (g_sem0, g_sem1)
    i_sems = (i_sem0, i_sem1)

    row0 = sub * ROWS_PT

    def zrow(r, c2):
        for j in range(D_NODE // 16):
            e_v[0, r, pl.ds(j * 16, 16)] = jnp.zeros((16,), jnp.float32)
        return c2

    lax.fori_loop(0, CH, zrow, 0)
    for b in range(ROWS_PT // CH):
        pltpu.sync_copy(e_v.at[0], aggr_sh.at[pl.ds(row0 + b * CH, CH)])
    plsc.subcore_barrier()

    base0 = core * EPC + sub * EPT

    def idx_load(ci, s):
        base = base0 + ci * CH
        pltpu.async_copy(src_hbm.at[pl.ds(base, CH)], src_vs[s], i_sems[s])
        pltpu.async_copy(dst_hbm.at[pl.ds(base, CH)], dst_vs[s], i_sems[s])

    def streams(ci, s):
        base = base0 + ci * CH
        pltpu.make_async_copy(src_hbm.at[pl.ds(base, CH)], src_vs[s],
                              i_sems[s]).wait()
        pltpu.make_async_copy(dst_hbm.at[pl.ds(base, CH)], dst_vs[s],
                              i_sems[s]).wait()
        pltpu.async_copy(e_hbm.at[pl.ds(base, CH)], e_v.at[s], e_sems[s])
        pltpu.async_copy(x_hbm.at[src_vs[s]], xr_v.at[s], g_sems[s])

    def process(ci, s):
        base = base0 + ci * CH
        pltpu.make_async_copy(e_hbm.at[pl.ds(base, CH)], e_v.at[s],
                              e_sems[s]).wait()
        pltpu.make_async_copy(x_hbm.at[src_vs[s]], xr_v.at[s],
                              g_sems[s]).wait()

        def row_body(r, c2):
            for j in range(D_NODE // 16):
                sl = pl.ds(j * 16, 16)
                e_v[s, r, sl] = jnp.maximum(e_v[s, r, sl] + xr_v[s, r, sl],
                                            0.0)
            return c2

        lax.fori_loop(0, CH, row_body, 0)
        pltpu.sync_copy(e_v.at[s], aggr_sh.at[dst_vs[s]], add=True)

    idx_load(0, 0)
    streams(0, 0)
    idx_load(1, 1)

    def pair_body(k2, c):
        i = 2 * k2
        streams(i + 1, 1)
        process(i, 0)
        idx_load(i + 2, 0)
        streams(i + 2, 0)
        process(i + 1, 1)
        idx_load(jnp.minimum(i + 3, NCH - 1), 1)
        return c

    lax.fori_loop(0, NCH // 2, pair_body, 0)
    process(NCH - 1, 0)
    pltpu.make_async_copy(src_hbm.at[pl.ds(base0, CH)], src_vs[1],
                          i_sems[1]).wait()
    pltpu.make_async_copy(dst_hbm.at[pl.ds(base0, CH)], dst_vs[1],
                          i_sems[1]).wait()
    plsc.subcore_barrier()

    pltpu.sync_copy(aggr_sh.at[pl.ds(row0, ROWS_PT)],
                    out_hbm.at[core, pl.ds(row0, ROWS_PT)])


def kernel(x, edge_index, edge_attr, We, be, W1, b1, W2, b2):
    ei = edge_index.astype(jnp.int32)

    eat = edge_attr.T
    e = pl.pallas_call(
        _proj_body,
        grid=(N_EDGES // 12800,),
        in_specs=[
            pl.BlockSpec((D_EDGE, 12800), lambda i: (0, i)),
            pl.BlockSpec((D_EDGE, D_NODE), lambda i: (0, 0)),
            pl.BlockSpec((1, D_NODE), lambda i: (0, 0)),
        ],
        out_specs=pl.BlockSpec((12800, D_NODE), lambda i: (i, 0)),
        out_shape=jax.ShapeDtypeStruct((N_EDGES, D_NODE), jnp.float32),
    )(eat, We, be.reshape(1, D_NODE))

    partials = pl.kernel(
        _sc_aggregate,
        mesh=plsc.VectorSubcoreMesh(core_axis_name="c", subcore_axis_name="s"),
        out_type=jax.ShapeDtypeStruct((NC, N_PAD, D_NODE), jnp.float32),
        scratch_types=[
            pltpu.VMEM((CH,), jnp.int32),
            pltpu.VMEM((CH,), jnp.int32),
            pltpu.VMEM((CH,), jnp.int32),
            pltpu.VMEM((CH,), jnp.int32),
            pltpu.VMEM((2, CH, D_NODE), jnp.float32),
            pltpu.VMEM((2, CH, D_NODE), jnp.float32),
            pltpu.VMEM_SHARED((N_PAD, D_NODE), jnp.float32),
            pltpu.SemaphoreType.DMA,
            pltpu.SemaphoreType.DMA,
            pltpu.SemaphoreType.DMA,
            pltpu.SemaphoreType.DMA,
            pltpu.SemaphoreType.DMA,
            pltpu.SemaphoreType.DMA,
        ],
    )(x, ei[0], ei[1], e)

    out = pl.pallas_call(
        _mlp_body,
        grid=(N_NODES // MLP_BLOCK,),
        in_specs=[
            pl.BlockSpec((MLP_BLOCK, D_NODE), lambda i: (i, 0)),
            pl.BlockSpec((NC, MLP_BLOCK, D_NODE), lambda i: (0, i, 0)),
            pl.BlockSpec((D_NODE, D_NODE), lambda i: (0, 0)),
            pl.BlockSpec((1, D_NODE), lambda i: (0, 0)),
            pl.BlockSpec((D_NODE, D_NODE), lambda i: (0, 0)),
            pl.BlockSpec((1, D_NODE), lambda i: (0, 0)),
        ],
        out_specs=pl.BlockSpec((MLP_BLOCK, D_NODE), lambda i: (i, 0)),
        out_shape=jax.ShapeDtypeStruct((N_NODES, D_NODE), jnp.float32),
    )(x, partials, W1, b1.reshape(1, D_NODE), W2, b2.reshape(1, D_NODE))
    return out
